# baseline (speedup 1.0000x reference)
import functools

import jax
import jax.numpy as jnp
from jax import lax
from jax.experimental import pallas as pl
from jax.experimental.pallas import tpu as pltpu
from jax.experimental.pallas import tpu_sc as plsc

N = 10000
D = 128
NC = 2
NS = 16
NW = NC * NS
K = 128
NPAD = 10112
RT = NPAD // NS
PAD_DST = N + 16
DD = 128


def _fill(buf, value):
    def body(r, carry):
        for q in range(D // 16):
            buf[r, pl.ds(q * 16, 16)] = jnp.full((16,), value, jnp.float32)
        return carry

    lax.fori_loop(0, K, body, 0)


def _zero_acc(zbuf, acc, s):
    base = s * RT
    full, rem = RT // K, RT % K
    for i in range(full):
        pltpu.sync_copy(zbuf, acc.at[pl.ds(base + i * K, K)])
    if rem:
        pltpu.sync_copy(zbuf.at[pl.ds(0, rem)],
                        acc.at[pl.ds(base + full * K, rem)])


def _deg_body(dst_hbm, out_hbm, idx_dst, ones_v, sem, acc, *, nb):
    c = lax.axis_index("c")
    s = lax.axis_index("s")
    tile = c * NS + s
    _fill(ones_v, 0.0)
    _zero_acc(ones_v, acc, s)
    _fill(ones_v, 1.0)
    pltpu.sync_copy(dst_hbm.at[pl.ds(tile * nb, nb)], idx_dst)
    plsc.subcore_barrier()

    def body(j, carry):
        pltpu.sync_copy(ones_v, acc.at[idx_dst.at[j]], add=True)
        return carry

    lax.fori_loop(0, nb, body, 0)
    plsc.subcore_barrier()
    pltpu.sync_copy(acc.at[pl.ds(s * RT, RT)],
                    out_hbm.at[c, pl.ds(s * RT, RT)])


def _spmm_body(y_hbm, src_hbm, dst_hbm, out_hbm,
               idx_src, idx_dst, rows0, rows1, sem, acc, *, nb):
    c = lax.axis_index("c")
    s = lax.axis_index("s")
    tile = c * NS + s
    half = nb // 2
    _fill(rows0, 0.0)
    _zero_acc(rows0, acc, s)
    plsc.subcore_barrier()

    for h in range(2):
        base = tile * nb + h * half
        pltpu.sync_copy(src_hbm.at[pl.ds(base, half)], idx_src)
        pltpu.sync_copy(dst_hbm.at[pl.ds(base, half)], idx_dst)
        pltpu.async_copy(y_hbm.at[idx_src.at[0]], rows0, sem).wait()

        def body(p, carry):
            for b, (buf, nxt) in enumerate(((rows0, rows1), (rows1, rows0))):
                j = p * 2 + b

                @pl.when(j + 1 < half)
                def _():
                    pltpu.async_copy(y_hbm.at[idx_src.at[j + 1]], nxt, sem)

                pltpu.sync_copy(buf, acc.at[idx_dst.at[j]], add=True)

                @pl.when(j + 1 < half)
                def _():
                    pltpu.make_async_copy(y_hbm.at[idx_src.at[j + 1]], nxt,
                                          sem).wait()
            return carry

        lax.fori_loop(0, half // 2, body, 0)
    plsc.subcore_barrier()
    pltpu.sync_copy(acc.at[pl.ds(s * RT, RT)],
                    out_hbm.at[c, pl.ds(s * RT, RT)])


def _make_sc_kernels(nb):
    mesh = plsc.VectorSubcoreMesh(core_axis_name="c", subcore_axis_name="s")
    deg = pl.kernel(
        functools.partial(_deg_body, nb=nb),
        out_type=jax.ShapeDtypeStruct((NC, NPAD, DD), jnp.float32),
        mesh=mesh,
        scratch_types=[
            pltpu.VMEM((nb, K), jnp.int32),
            pltpu.VMEM((K, DD), jnp.float32),
            pltpu.SemaphoreType.DMA,
            pltpu.VMEM_SHARED((NPAD, DD), jnp.float32),
        ],
    )
    spmm = pl.kernel(
        functools.partial(_spmm_body, nb=nb),
        out_type=jax.ShapeDtypeStruct((NC, NPAD, D), jnp.float32),
        mesh=mesh,
        scratch_types=[
            pltpu.VMEM((nb // 2, K), jnp.int32),
            pltpu.VMEM((nb // 2, K), jnp.int32),
            pltpu.VMEM((K, D), jnp.float32),
            pltpu.VMEM((K, D), jnp.float32),
            pltpu.SemaphoreType.DMA,
            pltpu.VMEM_SHARED((NPAD, D), jnp.float32),
        ],
    )
    return deg, spmm


def _prep_body(d0_ref, d1_ref, x_ref, y_ref, dinv_ref):
    deg = d0_ref[...][:, :1] + d1_ref[...][:, :1] + 1.0
    dv = lax.rsqrt(deg)
    dinv_ref[...] = dv
    y_ref[...] = x_ref[...] * dv


def _dense_body(s0_ref, s1_ref, y_ref, dinv_ref, w_ref, b_ref, out_ref,
                *, relu_scale):
    dv = dinv_ref[...]
    t = (s0_ref[...] + s1_ref[...] + y_ref[...]) * dv
    h = jnp.dot(t, w_ref[...], preferred_element_type=jnp.float32)
    h = h + b_ref[...]
    if relu_scale:
        h = jnp.maximum(h, 0.0) * dv
    out_ref[...] = h


_R = 1000


def _make_prep():
    return pl.pallas_call(
        _prep_body,
        grid=(N // _R,),
        in_specs=[
            pl.BlockSpec((_R, DD), lambda i: (i, 0)),
            pl.BlockSpec((_R, DD), lambda i: (i, 0)),
            pl.BlockSpec((_R, D), lambda i: (i, 0)),
        ],
        out_specs=[
            pl.BlockSpec((_R, D), lambda i: (i, 0)),
            pl.BlockSpec((_R, 1), lambda i: (i, 0)),
        ],
        out_shape=[
            jax.ShapeDtypeStruct((N, D), jnp.float32),
            jax.ShapeDtypeStruct((N, 1), jnp.float32),
        ],
    )


def _make_dense(relu_scale):
    return pl.pallas_call(
        functools.partial(_dense_body, relu_scale=relu_scale),
        grid=(N // _R,),
        in_specs=[
            pl.BlockSpec((_R, D), lambda i: (i, 0)),
            pl.BlockSpec((_R, D), lambda i: (i, 0)),
            pl.BlockSpec((_R, D), lambda i: (i, 0)),
            pl.BlockSpec((_R, 1), lambda i: (i, 0)),
            pl.BlockSpec((D, D), lambda i: (0, 0)),
            pl.BlockSpec((1, D), lambda i: (0, 0)),
        ],
        out_specs=pl.BlockSpec((_R, D), lambda i: (i, 0)),
        out_shape=jax.ShapeDtypeStruct((N, D), jnp.float32),
    )


def kernel(vert_features, edges, W1, b1, W2, b2):
    x = vert_features
    src = edges[0].astype(jnp.int32)
    dst = edges[1].astype(jnp.int32)
    e = src.shape[0]
    nb = -(-e // (NW * K))
    nb = ((nb + 7) // 8) * 8
    pad = NW * K * nb - e
    src_p = jnp.concatenate(
        [src, jnp.zeros((pad,), jnp.int32)]).reshape(NW * nb, K)
    dst_p = jnp.concatenate(
        [dst, jnp.full((pad,), PAD_DST, jnp.int32)]).reshape(NW * nb, K)

    deg_fn, spmm_fn = _make_sc_kernels(nb)
    prep_fn = _make_prep()
    dense1_fn = _make_dense(True)
    dense2_fn = _make_dense(False)

    degp = deg_fn(dst_p)
    y1, dinv = prep_fn(degp[0, :N], degp[1, :N], x)

    sp1 = spmm_fn(y1, src_p, dst_p)
    y2 = dense1_fn(sp1[0, :N], sp1[1, :N], y1, dinv,
                   W1, b1.reshape(1, D))

    sp2 = spmm_fn(y2, src_p, dst_p)
    out = dense2_fn(sp2[0, :N], sp2[1, :N], y2, dinv,
                    W2, b2.reshape(1, D))
    return out


# speedup vs baseline: 1.2363x; 1.2363x over previous
import functools

import jax
import jax.numpy as jnp
from jax import lax
from jax.experimental import pallas as pl
from jax.experimental.pallas import tpu as pltpu
from jax.experimental.pallas import tpu_sc as plsc

N = 10000
D = 128
NC = 2
NS = 16
NW = NC * NS
K = 128
NPAD = 10112
RT = NPAD // NS
PAD_DST = N + 16
DD = 128


def _deg_body(dst_hbm, zeros_hbm, ones_hbm, out_hbm,
              idx_dst, ones_v, sem, acc, *, nb):
    c = lax.axis_index("c")
    s = lax.axis_index("s")
    tile = c * NS + s
    pltpu.sync_copy(zeros_hbm, acc.at[pl.ds(s * RT, RT)])
    pltpu.sync_copy(ones_hbm, ones_v)
    pltpu.sync_copy(dst_hbm.at[pl.ds(tile * nb, nb)], idx_dst)
    plsc.subcore_barrier()

    def body(j, carry):
        pltpu.sync_copy(ones_v, acc.at[idx_dst.at[j]], add=True)
        return carry

    lax.fori_loop(0, nb, body, 0)
    plsc.subcore_barrier()
    pltpu.sync_copy(acc.at[pl.ds(s * RT, RT)],
                    out_hbm.at[c, pl.ds(s * RT, RT)])


def _spmm_body(y_hbm, src_hbm, dst_hbm, zeros_hbm, out_hbm,
               idx_src, idx_dst, rows0, rows1, sem, acc, *, nb0, nb1):
    c = lax.axis_index("c")
    s = lax.axis_index("s")
    pltpu.sync_copy(zeros_hbm, acc.at[pl.ds(s * RT, RT)])
    plsc.subcore_barrier()

    def run(blk0, nbc):
        half = nbc // 2
        for h in range(2):
            base = blk0 + h * half
            pltpu.sync_copy(src_hbm.at[pl.ds(base, half)],
                            idx_src.at[pl.ds(0, half)])
            pltpu.sync_copy(dst_hbm.at[pl.ds(base, half)],
                            idx_dst.at[pl.ds(0, half)])
            pltpu.async_copy(y_hbm.at[idx_src.at[0]], rows0, sem).wait()

            def body(p, carry):
                for b, (buf, nxt) in enumerate(((rows0, rows1),
                                                (rows1, rows0))):
                    j = p * 2 + b

                    @pl.when(j + 1 < half)
                    def _():
                        pltpu.async_copy(y_hbm.at[idx_src.at[j + 1]], nxt,
                                         sem)

                    pltpu.sync_copy(buf, acc.at[idx_dst.at[j]], add=True)

                    @pl.when(j + 1 < half)
                    def _():
                        pltpu.make_async_copy(y_hbm.at[idx_src.at[j + 1]],
                                              nxt, sem).wait()
                return carry

            lax.fori_loop(0, half // 2, body, 0)

    @pl.when(c == 0)
    def _():
        run(s * nb0, nb0)

    @pl.when(c == 1)
    def _():
        run(NS * nb0 + s * nb1, nb1)
    plsc.subcore_barrier()
    pltpu.sync_copy(acc.at[pl.ds(s * RT, RT)],
                    out_hbm.at[c, pl.ds(s * RT, RT)])


def _make_sc_kernels(nb, nb0, nb1):
    mesh = plsc.VectorSubcoreMesh(core_axis_name="c", subcore_axis_name="s")
    deg = pl.kernel(
        functools.partial(_deg_body, nb=nb),
        out_type=jax.ShapeDtypeStruct((NC, NPAD, DD), jnp.float32),
        mesh=mesh,
        scratch_types=[
            pltpu.VMEM((nb, K), jnp.int32),
            pltpu.VMEM((K, DD), jnp.float32),
            pltpu.SemaphoreType.DMA,
            pltpu.VMEM_SHARED((NPAD, DD), jnp.float32),
        ],
    )
    nbmax = max(nb0, nb1)
    spmm = pl.kernel(
        functools.partial(_spmm_body, nb0=nb0, nb1=nb1),
        out_type=jax.ShapeDtypeStruct((NC, NPAD, D), jnp.float32),
        mesh=mesh,
        scratch_types=[
            pltpu.VMEM((nbmax // 2, K), jnp.int32),
            pltpu.VMEM((nbmax // 2, K), jnp.int32),
            pltpu.VMEM((K, D), jnp.float32),
            pltpu.VMEM((K, D), jnp.float32),
            pltpu.SemaphoreType.DMA,
            pltpu.VMEM_SHARED((NPAD, D), jnp.float32),
        ],
    )
    return deg, spmm


def _prep_body(d0_ref, d1_ref, x_ref, y_ref, dinv_ref):
    deg = d0_ref[...][:, :1] + d1_ref[...][:, :1] + 1.0
    dv = lax.rsqrt(deg)
    dinv_ref[...] = dv
    y_ref[...] = x_ref[...] * dv


def _dense_body(s0_ref, s1_ref, y_ref, dinv_ref, w_ref, b_ref, out_ref,
                *, relu_scale):
    dv = dinv_ref[...]
    t = (s0_ref[...] + s1_ref[...] + y_ref[...]) * dv
    h = jnp.dot(t, w_ref[...], preferred_element_type=jnp.float32)
    h = h + b_ref[...]
    if relu_scale:
        h = jnp.maximum(h, 0.0) * dv
    out_ref[...] = h


_R = 1000


def _make_prep():
    return pl.pallas_call(
        _prep_body,
        grid=(N // _R,),
        in_specs=[
            pl.BlockSpec((_R, DD), lambda i: (i, 0)),
            pl.BlockSpec((_R, DD), lambda i: (i, 0)),
            pl.BlockSpec((_R, D), lambda i: (i, 0)),
        ],
        out_specs=[
            pl.BlockSpec((_R, D), lambda i: (i, 0)),
            pl.BlockSpec((_R, 1), lambda i: (i, 0)),
        ],
        out_shape=[
            jax.ShapeDtypeStruct((N, D), jnp.float32),
            jax.ShapeDtypeStruct((N, 1), jnp.float32),
        ],
    )


def _make_dense(relu_scale):
    return pl.pallas_call(
        functools.partial(_dense_body, relu_scale=relu_scale),
        grid=(N // _R,),
        in_specs=[
            pl.BlockSpec((_R, D), lambda i: (i, 0)),
            pl.BlockSpec((_R, D), lambda i: (i, 0)),
            pl.BlockSpec((_R, D), lambda i: (i, 0)),
            pl.BlockSpec((_R, 1), lambda i: (i, 0)),
            pl.BlockSpec((D, D), lambda i: (0, 0)),
            pl.BlockSpec((1, D), lambda i: (0, 0)),
        ],
        out_specs=pl.BlockSpec((_R, D), lambda i: (i, 0)),
        out_shape=jax.ShapeDtypeStruct((N, D), jnp.float32),
    )


def kernel(vert_features, edges, W1, b1, W2, b2):
    x = vert_features
    src = edges[0].astype(jnp.int32)
    dst = edges[1].astype(jnp.int32)
    e = src.shape[0]
    nb = -(-e // (NW * K))
    nb = ((nb + 7) // 8) * 8
    pad = NW * K * nb - e
    src_p = jnp.concatenate(
        [src, jnp.zeros((pad,), jnp.int32)]).reshape(NW * nb, K)
    dst_p = jnp.concatenate(
        [dst, jnp.full((pad,), PAD_DST, jnp.int32)]).reshape(NW * nb, K)

    t_blocks = 2 * nb
    nb0 = (7 * t_blocks // 10 // 16) * 16
    nb1 = t_blocks - nb0

    zeros_deg = jnp.zeros((RT, DD), jnp.float32)
    ones_deg = jnp.ones((K, DD), jnp.float32)
    zeros_rows = jnp.zeros((RT, D), jnp.float32)

    deg_fn, spmm_fn = _make_sc_kernels(nb, nb0, nb1)
    prep_fn = _make_prep()
    dense1_fn = _make_dense(True)
    dense2_fn = _make_dense(False)

    degp = deg_fn(dst_p, zeros_deg, ones_deg)
    y1, dinv = prep_fn(degp[0, :N], degp[1, :N], x)

    sp1 = spmm_fn(y1, src_p, dst_p, zeros_rows)
    y2 = dense1_fn(sp1[0, :N], sp1[1, :N], y1, dinv,
                   W1, b1.reshape(1, D))

    sp2 = spmm_fn(y2, src_p, dst_p, zeros_rows)
    out = dense2_fn(sp2[0, :N], sp2[1, :N], y2, dinv,
                    W2, b2.reshape(1, D))
    return out


# speedup vs baseline: 1.3301x; 1.0759x over previous
import functools

import jax
import jax.numpy as jnp
from jax import lax
from jax.experimental import pallas as pl
from jax.experimental.pallas import tpu as pltpu
from jax.experimental.pallas import tpu_sc as plsc

N = 10000
D = 128
NC = 2
NS = 16
NW = NC * NS
K = 128
NPAD = 10112
RT = NPAD // NS
PAD_DST = N + 16
DD = 128
CHUNK = 48


def _deg_body(dst_hbm, zeros_hbm, ones_hbm, out_hbm,
              idx_dst, ones_v, sem, acc, *, nb):
    c = lax.axis_index("c")
    s = lax.axis_index("s")
    tile = c * NS + s
    pltpu.sync_copy(zeros_hbm, acc.at[pl.ds(s * RT, RT)])
    pltpu.sync_copy(ones_hbm, ones_v)
    pltpu.sync_copy(dst_hbm.at[pl.ds(tile * nb, nb)], idx_dst)
    plsc.subcore_barrier()

    def body(j, carry):
        pltpu.sync_copy(ones_v, acc.at[idx_dst.at[j]], add=True)
        return carry

    lax.fori_loop(0, nb, body, 0)
    plsc.subcore_barrier()
    pltpu.sync_copy(acc.at[pl.ds(s * RT, RT)],
                    out_hbm.at[c, pl.ds(s * RT, RT)])


def _spmm_body(y_hbm, src_hbm, dst_hbm, zeros_hbm, out_hbm,
               idx_src, idx_dst, rows0, rows1, sem, acc, *, nb0, nb1):
    c = lax.axis_index("c")
    s = lax.axis_index("s")
    pltpu.sync_copy(zeros_hbm, acc.at[pl.ds(s * RT, RT)])
    plsc.subcore_barrier()

    def run(blk0, nbc):
        base = blk0
        rem = nbc
        while rem > 0:
            ch = min(CHUNK, rem)
            pltpu.sync_copy(src_hbm.at[pl.ds(base, ch)],
                            idx_src.at[pl.ds(0, ch)])
            pltpu.sync_copy(dst_hbm.at[pl.ds(base, ch)],
                            idx_dst.at[pl.ds(0, ch)])
            pltpu.async_copy(y_hbm.at[idx_src.at[0]], rows0, sem).wait()

            def body(p, carry):
                for b, (buf, nxt) in enumerate(((rows0, rows1),
                                                (rows1, rows0))):
                    j = p * 2 + b

                    @pl.when(j + 1 < ch)
                    def _():
                        pltpu.async_copy(y_hbm.at[idx_src.at[j + 1]], nxt,
                                         sem)

                    pltpu.sync_copy(buf, acc.at[idx_dst.at[j]], add=True)

                    @pl.when(j + 1 < ch)
                    def _():
                        pltpu.make_async_copy(y_hbm.at[idx_src.at[j + 1]],
                                              nxt, sem).wait()
                return carry

            lax.fori_loop(0, ch // 2, body, 0)
            base += ch
            rem -= ch

    @pl.when(c == 0)
    def _():
        run(s * nb0, nb0)

    @pl.when(c == 1)
    def _():
        run(NS * nb0 + s * nb1, nb1)
    plsc.subcore_barrier()
    pltpu.sync_copy(acc.at[pl.ds(s * RT, RT)],
                    out_hbm.at[c, pl.ds(s * RT, RT)])


def _make_sc_kernels(nb, nb0, nb1):
    mesh = plsc.VectorSubcoreMesh(core_axis_name="c", subcore_axis_name="s")
    deg = pl.kernel(
        functools.partial(_deg_body, nb=nb),
        out_type=jax.ShapeDtypeStruct((NC, NPAD, DD), jnp.float32),
        mesh=mesh,
        scratch_types=[
            pltpu.VMEM((nb, K), jnp.int32),
            pltpu.VMEM((K, DD), jnp.float32),
            pltpu.SemaphoreType.DMA,
            pltpu.VMEM_SHARED((NPAD, DD), jnp.float32),
        ],
    )
    spmm = pl.kernel(
        functools.partial(_spmm_body, nb0=nb0, nb1=nb1),
        out_type=jax.ShapeDtypeStruct((NC, NPAD, D), jnp.float32),
        mesh=mesh,
        scratch_types=[
            pltpu.VMEM((CHUNK, K), jnp.int32),
            pltpu.VMEM((CHUNK, K), jnp.int32),
            pltpu.VMEM((K, D), jnp.float32),
            pltpu.VMEM((K, D), jnp.float32),
            pltpu.SemaphoreType.DMA,
            pltpu.VMEM_SHARED((NPAD, D), jnp.float32),
        ],
    )
    return deg, spmm


def _prep_body(d0_ref, d1_ref, x_ref, y_ref, dinv_ref):
    deg = d0_ref[...][:, :1] + d1_ref[...][:, :1] + 1.0
    dv = lax.rsqrt(deg)
    dinv_ref[...] = dv
    y_ref[...] = x_ref[...] * dv


def _dense_body(s0_ref, s1_ref, y_ref, dinv_ref, w_ref, b_ref, out_ref,
                *, relu_scale):
    dv = dinv_ref[...]
    t = (s0_ref[...] + s1_ref[...] + y_ref[...]) * dv
    h = jnp.dot(t, w_ref[...], preferred_element_type=jnp.float32)
    h = h + b_ref[...]
    if relu_scale:
        h = jnp.maximum(h, 0.0) * dv
    out_ref[...] = h


_R = 1000


def _make_prep():
    return pl.pallas_call(
        _prep_body,
        grid=(N // _R,),
        in_specs=[
            pl.BlockSpec((_R, DD), lambda i: (i, 0)),
            pl.BlockSpec((_R, DD), lambda i: (i, 0)),
            pl.BlockSpec((_R, D), lambda i: (i, 0)),
        ],
        out_specs=[
            pl.BlockSpec((_R, D), lambda i: (i, 0)),
            pl.BlockSpec((_R, 1), lambda i: (i, 0)),
        ],
        out_shape=[
            jax.ShapeDtypeStruct((N, D), jnp.float32),
            jax.ShapeDtypeStruct((N, 1), jnp.float32),
        ],
    )


def _make_dense(relu_scale):
    return pl.pallas_call(
        functools.partial(_dense_body, relu_scale=relu_scale),
        grid=(N // _R,),
        in_specs=[
            pl.BlockSpec((_R, D), lambda i: (i, 0)),
            pl.BlockSpec((_R, D), lambda i: (i, 0)),
            pl.BlockSpec((_R, D), lambda i: (i, 0)),
            pl.BlockSpec((_R, 1), lambda i: (i, 0)),
            pl.BlockSpec((D, D), lambda i: (0, 0)),
            pl.BlockSpec((1, D), lambda i: (0, 0)),
        ],
        out_specs=pl.BlockSpec((_R, D), lambda i: (i, 0)),
        out_shape=jax.ShapeDtypeStruct((N, D), jnp.float32),
    )


def kernel(vert_features, edges, W1, b1, W2, b2):
    x = vert_features
    src = edges[0].astype(jnp.int32)
    dst = edges[1].astype(jnp.int32)
    e = src.shape[0]
    nb = -(-e // (NW * K))
    nb = ((nb + 7) // 8) * 8
    pad = NW * K * nb - e
    src_p = jnp.concatenate(
        [src, jnp.zeros((pad,), jnp.int32)]).reshape(NW * nb, K)
    dst_p = jnp.concatenate(
        [dst, jnp.full((pad,), PAD_DST, jnp.int32)]).reshape(NW * nb, K)

    t_blocks = 2 * nb
    nb0 = (9 * t_blocks // 10 // 16) * 16
    nb1 = t_blocks - nb0

    zeros_deg = jnp.zeros((RT, DD), jnp.float32)
    ones_deg = jnp.ones((K, DD), jnp.float32)
    zeros_rows = jnp.zeros((RT, D), jnp.float32)

    deg_fn, spmm_fn = _make_sc_kernels(nb, nb0, nb1)
    prep_fn = _make_prep()
    dense1_fn = _make_dense(True)
    dense2_fn = _make_dense(False)

    degp = deg_fn(dst_p, zeros_deg, ones_deg)
    y1, dinv = prep_fn(degp[0, :N], degp[1, :N], x)

    sp1 = spmm_fn(y1, src_p, dst_p, zeros_rows)
    y2 = dense1_fn(sp1[0, :N], sp1[1, :N], y1, dinv,
                   W1, b1.reshape(1, D))

    sp2 = spmm_fn(y2, src_p, dst_p, zeros_rows)
    out = dense2_fn(sp2[0, :N], sp2[1, :N], y2, dinv,
                    W2, b2.reshape(1, D))
    return out
